# baseline (speedup 1.0000x reference)
import functools

import jax
import jax.numpy as jnp
from jax import lax
from jax.experimental import pallas as pl
from jax.experimental.pallas import tpu as pltpu
from jax.experimental.pallas import tpu_sc as plsc

F32 = jnp.float32


def _dot(a, b):
    return jnp.dot(a, b, preferred_element_type=F32,
                   precision=lax.Precision.HIGHEST)


def _lra_call(x, lw, pre_w=None):
    n, d = x.shape
    k = lw['u'].shape[1]

    def body(x_ref, u_ref, v_ref, z_ref, t_ref, *rest):
        x_ = x_ref[...]
        u = _dot(x_, u_ref[...])
        v = _dot(x_, v_ref[...])
        z = _dot(x_, z_ref[...])
        t = _dot(x_, t_ref[...])
        m = lax.dot_general(v, z, (((0,), (0,)), ((), ())),
                            preferred_element_type=F32)
        dm = _dot(u, m) * (1.0 / n)
        if pre_w is None:
            (att_ref,) = rest
        else:
            w_ref, att_ref, pre_ref = rest
            pre_ref[...] = _dot(x_, w_ref[...])
        att_ref[...] = jnp.concatenate([dm, t], axis=1)

    out_shape = [jax.ShapeDtypeStruct((n, 2 * k), F32)]
    operands = [x, lw['u'], lw['v'], lw['z'], lw['t']]
    if pre_w is not None:
        operands.append(pre_w)
        out_shape.append(jax.ShapeDtypeStruct((n, pre_w.shape[1]), F32))
    outs = pl.pallas_call(body, out_shape=out_shape)(*operands)
    return outs if pre_w is not None else outs[0]


def _layer_call(kind, hd, hp, s, att, wdd, wpd, wpp=None, wdp=None, br=1000):
    n, dd = hd.shape
    grid = (n // br,)
    nc = s.shape[0]
    ko = nc - 2
    wpd_r = wpd.reshape(2, 128, wpd.shape[1])
    d1 = dd + att.shape[1]
    d1pad = (-(-d1 // 128) * 128) if kind == 'last' else d1

    def body(*refs):
        if kind == 'first':
            (hd_ref, hp_ref, s_ref, att_ref, wdd_ref, wpd_ref, wpp_ref,
             wdp_ref, hdo_ref, hpo_ref, x_ref) = refs
        elif kind == 'mid':
            (hd_ref, hp_ref, s_ref, att_ref, wdd_ref, wpd_ref, wpp_ref,
             hdo_ref, x_ref) = refs
        else:
            hd_ref, s_ref, att_ref, wdd_ref, wpd_ref, hdo_ref = refs
        hd_ = hd_ref[...]
        msg_d = _dot(s_ref[ko], wpd_ref[0]) + _dot(s_ref[ko + 1], wpd_ref[1])
        hd_c = _dot(hd_, wdd_ref[...]) + msg_d
        if kind != 'first':
            hd_c = hd_c + hd_
        parts = [jnp.maximum(hd_c, 0.0), jnp.maximum(att_ref[...], 0.0)]
        if kind == 'last' and d1pad > d1:
            parts.append(jnp.zeros((hd_.shape[0], d1pad - d1), F32))
        h = jnp.concatenate(parts, axis=1)
        hdo_ref[...] = h
        if kind != 'last':
            hp_ = hp_ref[...]
            hp_c = (_dot(hp_, wpp_ref[...])
                    + jnp.concatenate([s_ref[0], s_ref[1]], axis=1))
            if kind != 'first':
                hp_c = hp_c + hp_
            hpr = jnp.maximum(hp_c, 0.0)
            if kind == 'first':
                hpo_ref[...] = hpr
                xp = _dot(h, wdp_ref[...])
                x_ref[0] = xp[:, :128]
                x_ref[1] = xp[:, 128:]
                x_ref[2] = hpr[:, :128]
                x_ref[3] = hpr[:, 128:]
            else:
                x_ref[0] = hpr[:, :128]
                x_ref[1] = hpr[:, 128:]

    row = lambda w: pl.BlockSpec((br, w), lambda i: (i, 0))
    full = lambda a: pl.BlockSpec(a.shape, lambda i: tuple(0 for _ in a.shape))
    s_spec = pl.BlockSpec((nc, br, 128), lambda i: (0, i, 0))
    x_spec = pl.BlockSpec((4 if kind == 'first' else 2, br, 128),
                          lambda i: (0, i, 0))

    if kind == 'first':
        operands = [hd, hp, s, att, wdd, wpd_r, wpp, wdp]
        in_specs = [row(dd), row(256), s_spec, row(att.shape[1]),
                    full(wdd), full(wpd_r), full(wpp), full(wdp)]
        out_shape = [jax.ShapeDtypeStruct((n, d1), F32),
                     jax.ShapeDtypeStruct((n, 256), F32),
                     jax.ShapeDtypeStruct((4, n, 128), F32)]
        out_specs = [row(d1), row(256), x_spec]
    elif kind == 'mid':
        operands = [hd, hp, s, att, wdd, wpd_r, wpp]
        in_specs = [row(dd), row(256), s_spec, row(att.shape[1]),
                    full(wdd), full(wpd_r), full(wpp)]
        out_shape = [jax.ShapeDtypeStruct((n, d1), F32),
                     jax.ShapeDtypeStruct((2, n, 128), F32)]
        out_specs = [row(d1), x_spec]
    else:
        operands = [hd, s, att, wdd, wpd_r]
        in_specs = [row(dd), s_spec, row(att.shape[1]), full(wdd), full(wpd_r)]
        out_shape = jax.ShapeDtypeStruct((n, d1pad), F32)
        out_specs = row(d1pad)

    return pl.pallas_call(body, grid=grid, in_specs=in_specs,
                          out_specs=out_specs, out_shape=out_shape)(*operands)


def _pred_call(g, attr, w1a, w1b, w1c, b1, w2, b2, br=512):
    nb, aw = attr.shape
    dcols = g.shape[1]
    grid = (nb // br,)

    def body(hi_ref, hj_ref, at_ref, w1a_ref, w1b_ref, w1c_ref, b1_ref,
             w2_ref, b2_ref, o_ref):
        z = (_dot(hi_ref[...], w1a_ref[...])
             + _dot(hj_ref[...], w1b_ref[...])
             + _dot(at_ref[...], w1c_ref[...]) + b1_ref[...])
        z = jnp.maximum(z, 0.0)
        o_ref[...] = _dot(z, w2_ref[...]) + b2_ref[...]

    off = nb // br
    full = lambda a: pl.BlockSpec(a.shape, lambda i: tuple(0 for _ in a.shape))
    in_specs = [
        pl.BlockSpec((br, dcols), lambda i: (i, 0)),
        pl.BlockSpec((br, dcols), lambda i: (i + off, 0)),
        pl.BlockSpec((br, aw), lambda i: (i, 0)),
        full(w1a), full(w1b), full(w1c), full(b1), full(w2), full(b2),
    ]
    out = pl.pallas_call(
        body, grid=grid, in_specs=in_specs,
        out_specs=pl.BlockSpec((br, 1), lambda i: (i, 0)),
        out_shape=jax.ShapeDtypeStruct((nb, 1), F32),
    )(g, g, attr, w1a, w1b, w1c, b1, w2, b2)
    return out[:, 0]


_NC, _NS = 2, 16


def _sc_scatter(x, ei, zeros_slice, nchunk):
    _, n, _ = x.shape
    nbt = ei.shape[0]
    nblk = nbt // _NS
    assert nbt % _NS == 0 and nblk % 2 == 0
    nhalf = nblk // 2
    rpt = (-(-n // _NS) + 7) // 8 * 8
    passes = nchunk // _NC
    mesh = plsc.VectorSubcoreMesh(core_axis_name="c", subcore_axis_name="s")

    @functools.partial(
        pl.kernel,
        out_type=jax.ShapeDtypeStruct((nchunk, n, 128), F32),
        mesh=mesh,
        scratch_types=[
            pltpu.VMEM((2, 128), jnp.int32),
            pltpu.VMEM((2, 128), jnp.int32),
            pltpu.VMEM((128, 128), F32),
            pltpu.VMEM((128, 128), F32),
            pltpu.VMEM_SHARED((n + 8, 128), F32),
            pltpu.SemaphoreType.DMA,
            pltpu.SemaphoreType.DMA,
            pltpu.SemaphoreType.DMA,
            pltpu.SemaphoreType.DMA,
        ],
    )
    def k(x_hbm, ei_hbm, z_hbm, out_hbm, ib0, ib1, rows0, rows1, acc,
          semi0, semi1, semg0, semg1):
        c = lax.axis_index("c")
        s = lax.axis_index("s")
        r0 = pl.multiple_of(jnp.minimum(s * rpt, n - rpt), 8)
        base = s * nblk
        for p in range(passes):
            chunk = p * _NC + c
            pltpu.sync_copy(z_hbm, acc.at[pl.ds(r0, rpt)])
            plsc.subcore_barrier()
            xc = x_hbm.at[chunk]
            pltpu.async_copy(ei_hbm.at[base], ib0, semi0)

            def body(i, _):
                g0 = base + 2 * i
                pltpu.make_async_copy(ei_hbm.at[base], ib0, semi0).wait()
                pltpu.async_copy(xc.at[ib0.at[0]], rows0, semg0)

                @pl.when(i > 0)
                def _prev():
                    pltpu.make_async_copy(xc.at[ib1.at[0]], rows1,
                                          semg1).wait()
                    pltpu.sync_copy(rows1, acc.at[ib1.at[1]], add=True)

                pltpu.async_copy(ei_hbm.at[g0 + 1], ib1, semi1)
                pltpu.make_async_copy(ei_hbm.at[base], ib1, semi1).wait()
                pltpu.make_async_copy(xc.at[ib0.at[0]], rows0, semg0).wait()
                pltpu.async_copy(xc.at[ib1.at[0]], rows1, semg1)
                pltpu.sync_copy(rows0, acc.at[ib0.at[1]], add=True)

                @pl.when(i + 1 < nhalf)
                def _next():
                    pltpu.async_copy(ei_hbm.at[g0 + 2], ib0, semi0)

                return _

            lax.fori_loop(0, nhalf, body, None)
            pltpu.make_async_copy(xc.at[ib1.at[0]], rows1, semg1).wait()
            pltpu.sync_copy(rows1, acc.at[ib1.at[1]], add=True)
            plsc.subcore_barrier()
            pltpu.sync_copy(acc.at[pl.ds(r0, rpt)],
                            out_hbm.at[chunk, pl.ds(r0, rpt)])
            plsc.subcore_barrier()

    return k(x, ei, zeros_slice)


def _sc_gather(tab, idx):
    n, dcols = tab.shape
    ni = idx.shape[0]
    per_w = ni // (_NC * _NS)
    nb = per_w // 128
    mesh = plsc.VectorSubcoreMesh(core_axis_name="c", subcore_axis_name="s")

    @functools.partial(
        pl.kernel,
        out_type=jax.ShapeDtypeStruct((ni, dcols), F32),
        mesh=mesh,
        scratch_types=[
            pltpu.VMEM((128,), jnp.int32),
            pltpu.VMEM((128, dcols), F32),
            pltpu.SemaphoreType.DMA,
        ],
    )
    def k(tab_hbm, idx_hbm, out_hbm, ibuf, rows, sem):
        c = lax.axis_index("c")
        s = lax.axis_index("s")
        wid = s * _NC + c
        for b in range(nb):
            o = pl.multiple_of(wid * per_w + b * 128, 128)
            pltpu.sync_copy(idx_hbm.at[pl.ds(o, 128)], ibuf)
            pltpu.async_copy(tab_hbm.at[ibuf], rows, sem).wait()
            pltpu.sync_copy(rows, out_hbm.at[pl.ds(o, 128)])

    return k(tab, idx)


def kernel(x_drugs, ddi_edge_attr, params, edge_index, ddi_edge_idx):
    nd, d = x_drugs.shape
    prot = params['prot_emb']
    npr = prot.shape[0]
    n = nd + npr
    c1, r0, r1 = params['conv1'], params['res0'], params['res1']
    zeros_slice = jnp.zeros(((-(-n // _NS) + 7) // 8 * 8, 128), F32)

    e = edge_index.shape[1]
    epad = -(-e // (_NS * 256)) * (_NS * 256)
    if epad != e:
        pn = epad - e
        pad_edges = jnp.stack([
            jnp.zeros((pn,), jnp.int32),
            n + (jnp.arange(pn, dtype=jnp.int32) % 8)])
        edge_index = jnp.concatenate([edge_index, pad_edges], axis=1)
    edge_index = edge_index.reshape(2, epad // 128, 128).transpose(1, 0, 2)

    def pad(a):
        return jnp.concatenate(
            [a, jnp.zeros((n - a.shape[0], a.shape[1]), F32)], axis=0)

    hd0 = pad(x_drugs)
    hp0 = jnp.concatenate([jnp.zeros((nd, d), F32), prot], axis=0)

    att0, xdp1 = _lra_call(x_drugs, params['lra0'], pre_w=c1['dp'])
    xdp1p = pad(xdp1)
    x1 = jnp.stack([xdp1p[:, :128], xdp1p[:, 128:],
                    hp0[:, :128], hp0[:, 128:]])
    s1 = _sc_scatter(x1, edge_index, zeros_slice, 4)
    hd1, hp1, x2 = _layer_call('first', hd0, hp0, s1, pad(att0),
                               wdd=c1['dd'], wpd=c1['pd'], wpp=c1['pp'],
                               wdp=r0['dp'])

    s2 = _sc_scatter(x2, edge_index, zeros_slice, 4)
    att1 = _lra_call(hd1[:nd], params['lra1'])
    hd2, x3 = _layer_call('mid', hd1, hp1, s2, pad(att1),
                          wdd=r0['dd'], wpd=r0['pd'], wpp=r0['pp'])

    s3 = _sc_scatter(x3, edge_index, zeros_slice, 2)
    att2 = _lra_call(hd2[:nd], params['lra2'])
    hd3 = _layer_call('last', hd2, None, s3, pad(att2),
                      wdd=r1['dd'], wpd=r1['pd'])

    idx_flat = jnp.concatenate([ddi_edge_idx[:, 0], ddi_edge_idx[:, 1]])
    g = _sc_gather(hd3, idx_flat)
    w1 = params['pred']['W1']
    d3 = hd2.shape[1] + 2 * params['lra2']['u'].shape[1]
    dpad = hd3.shape[1] - d3
    wpad = lambda w: jnp.concatenate(
        [w, jnp.zeros((dpad, w.shape[1]), F32)], axis=0)
    return _pred_call(g, ddi_edge_attr,
                      wpad(w1[:d3]), wpad(w1[d3:2 * d3]), w1[2 * d3:],
                      params['pred']['b1'].reshape(1, -1),
                      params['pred']['W2'],
                      params['pred']['b2'].reshape(1, 1))


# speedup vs baseline: 1.2034x; 1.2034x over previous
import functools

import jax
import jax.numpy as jnp
from jax import lax
from jax.experimental import pallas as pl
from jax.experimental.pallas import tpu as pltpu
from jax.experimental.pallas import tpu_sc as plsc

F32 = jnp.float32


def _dot(a, b, precision=None):
    return jnp.dot(a, b, preferred_element_type=F32, precision=precision)


def _lra_call(x, lw, pre_w=None):
    n, d = x.shape
    k = lw['u'].shape[1]

    def body(x_ref, u_ref, v_ref, z_ref, t_ref, *rest):
        x_ = x_ref[...]
        u = _dot(x_, u_ref[...])
        v = _dot(x_, v_ref[...])
        z = _dot(x_, z_ref[...])
        t = _dot(x_, t_ref[...])
        m = lax.dot_general(v, z, (((0,), (0,)), ((), ())),
                            preferred_element_type=F32)
        dm = _dot(u, m) * (1.0 / n)
        if pre_w is None:
            (att_ref,) = rest
        else:
            w_ref, att_ref, pre_ref = rest
            pre_ref[...] = _dot(x_, w_ref[...])
        att_ref[...] = jnp.concatenate([dm, t], axis=1)

    out_shape = [jax.ShapeDtypeStruct((n, 2 * k), F32)]
    operands = [x, lw['u'], lw['v'], lw['z'], lw['t']]
    if pre_w is not None:
        operands.append(pre_w)
        out_shape.append(jax.ShapeDtypeStruct((n, pre_w.shape[1]), F32))
    outs = pl.pallas_call(body, out_shape=out_shape)(*operands)
    return outs if pre_w is not None else outs[0]


def _mm_call(a, w, br=1000):
    n, ka = a.shape

    def body(a_ref, w_ref, o_ref):
        o_ref[...] = _dot(a_ref[...], w_ref[...])

    return pl.pallas_call(
        body, grid=(n // br,),
        in_specs=[pl.BlockSpec((br, ka), lambda i: (i, 0)),
                  pl.BlockSpec(w.shape, lambda i: (0, 0))],
        out_specs=pl.BlockSpec((br, w.shape[1]), lambda i: (i, 0)),
        out_shape=jax.ShapeDtypeStruct((n, w.shape[1]), F32))(a, w)


def _layer_call(kind, hd, hp, s, att, wdd, wpd, wpp=None, wdp=None, br=1000):
    n, dd = hd.shape
    grid = (n // br,)
    nc = s.shape[0]
    ko = nc - 2
    wpd_r = None if wpd is None else wpd.reshape(2, 128, wpd.shape[1])
    d1 = dd + att.shape[1]
    d1pad = (-(-d1 // 128) * 128) if kind == 'last' else d1

    def body(*refs):
        if kind == 'first':
            (hd_ref, hp_ref, s_ref, att_ref, wdd_ref, wpp_ref,
             wdp_ref, hdo_ref, hpo_ref, x_ref) = refs
            wpd_ref = None
        elif kind == 'mid':
            (hd_ref, hp_ref, s_ref, att_ref, wdd_ref, wpd_ref, wpp_ref,
             hdo_ref, x_ref) = refs
        else:
            hd_ref, s_ref, att_ref, wdd_ref, wpd_ref, hdo_ref = refs
        hd_ = hd_ref[...]
        if kind == 'first':
            msg_d = jnp.concatenate([s_ref[ko], s_ref[ko + 1]], axis=1)
        else:
            msg_d = (_dot(s_ref[ko], wpd_ref[0])
                     + _dot(s_ref[ko + 1], wpd_ref[1]))
        hd_c = _dot(hd_, wdd_ref[...]) + msg_d
        if kind != 'first':
            hd_c = hd_c + hd_
        parts = [jnp.maximum(hd_c, 0.0), jnp.maximum(att_ref[...], 0.0)]
        if kind == 'last' and d1pad > d1:
            parts.append(jnp.zeros((hd_.shape[0], d1pad - d1), F32))
        h = jnp.concatenate(parts, axis=1)
        hdo_ref[...] = h
        if kind != 'last':
            hp_ = hp_ref[...]
            hp_c = (_dot(hp_, wpp_ref[...])
                    + jnp.concatenate([s_ref[0], s_ref[1]], axis=1))
            if kind != 'first':
                hp_c = hp_c + hp_
            hpr = jnp.maximum(hp_c, 0.0)
            if kind == 'first':
                hpo_ref[...] = hpr
                xp = _dot(h, wdp_ref[...])
                x_ref[0] = xp[:, :128]
                x_ref[1] = xp[:, 128:]
                x_ref[2] = hpr[:, :128]
                x_ref[3] = hpr[:, 128:]
            else:
                x_ref[0] = hpr[:, :128]
                x_ref[1] = hpr[:, 128:]

    row = lambda w: pl.BlockSpec((br, w), lambda i: (i, 0))
    full = lambda a: pl.BlockSpec(a.shape, lambda i: tuple(0 for _ in a.shape))
    s_spec = pl.BlockSpec((nc, br, 128), lambda i: (0, i, 0))
    x_spec = pl.BlockSpec((4 if kind == 'first' else 2, br, 128),
                          lambda i: (0, i, 0))

    if kind == 'first':
        operands = [hd, hp, s, att, wdd, wpp, wdp]
        in_specs = [row(dd), row(256), s_spec, row(att.shape[1]),
                    full(wdd), full(wpp), full(wdp)]
        out_shape = [jax.ShapeDtypeStruct((n, d1), F32),
                     jax.ShapeDtypeStruct((n, 256), F32),
                     jax.ShapeDtypeStruct((4, n, 128), F32)]
        out_specs = [row(d1), row(256), x_spec]
    elif kind == 'mid':
        operands = [hd, hp, s, att, wdd, wpd_r, wpp]
        in_specs = [row(dd), row(256), s_spec, row(att.shape[1]),
                    full(wdd), full(wpd_r), full(wpp)]
        out_shape = [jax.ShapeDtypeStruct((n, d1), F32),
                     jax.ShapeDtypeStruct((2, n, 128), F32)]
        out_specs = [row(d1), x_spec]
    else:
        operands = [hd, s, att, wdd, wpd_r]
        in_specs = [row(dd), s_spec, row(att.shape[1]), full(wdd),
                    full(wpd_r)]
        out_shape = jax.ShapeDtypeStruct((n, d1pad), F32)
        out_specs = row(d1pad)

    return pl.pallas_call(body, grid=grid, in_specs=in_specs,
                          out_specs=out_specs, out_shape=out_shape)(*operands)


def _pred_call(g, attr, w1a, w1b, w1c, b1, w2, b2, br=512):
    nb, aw = attr.shape
    dcols = g.shape[1]
    grid = (nb // br,)

    def body(hi_ref, hj_ref, at_ref, w1a_ref, w1b_ref, w1c_ref, b1_ref,
             w2_ref, b2_ref, o_ref):
        z = (_dot(hi_ref[...], w1a_ref[...])
             + _dot(hj_ref[...], w1b_ref[...])
             + _dot(at_ref[...], w1c_ref[...]) + b1_ref[...])
        z = jnp.maximum(z, 0.0)
        o_ref[...] = _dot(z, w2_ref[...]) + b2_ref[...]

    off = nb // br
    full = lambda a: pl.BlockSpec(a.shape, lambda i: tuple(0 for _ in a.shape))
    in_specs = [
        pl.BlockSpec((br, dcols), lambda i: (i, 0)),
        pl.BlockSpec((br, dcols), lambda i: (i + off, 0)),
        pl.BlockSpec((br, aw), lambda i: (i, 0)),
        full(w1a), full(w1b), full(w1c), full(b1), full(w2), full(b2),
    ]
    out = pl.pallas_call(
        body, grid=grid, in_specs=in_specs,
        out_specs=pl.BlockSpec((br, 1), lambda i: (i, 0)),
        out_shape=jax.ShapeDtypeStruct((nb, 1), F32),
    )(g, g, attr, w1a, w1b, w1c, b1, w2, b2)
    return out[:, 0]


_NC, _NS = 2, 16


def _sc_scatter(x, ei, zeros_slice, nchunk):
    _, n, _ = x.shape
    nbt = ei.shape[0]
    nblk = nbt // _NS
    assert nbt % _NS == 0 and nblk % 2 == 0
    nhalf = nblk // 2
    rpt = (-(-n // _NS) + 7) // 8 * 8
    passes = -(-nchunk // _NC)
    mesh = plsc.VectorSubcoreMesh(core_axis_name="c", subcore_axis_name="s")

    @functools.partial(
        pl.kernel,
        out_type=jax.ShapeDtypeStruct((nchunk, n, 128), F32),
        mesh=mesh,
        scratch_types=[
            pltpu.VMEM((2, 128), jnp.int32),
            pltpu.VMEM((2, 128), jnp.int32),
            pltpu.VMEM((128, 128), F32),
            pltpu.VMEM((128, 128), F32),
            pltpu.VMEM_SHARED((n + 8, 128), F32),
            pltpu.SemaphoreType.DMA,
            pltpu.SemaphoreType.DMA,
            pltpu.SemaphoreType.DMA,
            pltpu.SemaphoreType.DMA,
        ],
    )
    def k(x_hbm, ei_hbm, z_hbm, out_hbm, ib0, ib1, rows0, rows1, acc,
          semi0, semi1, semg0, semg1):
        c = lax.axis_index("c")
        s = lax.axis_index("s")
        r0 = pl.multiple_of(jnp.minimum(s * rpt, n - rpt), 8)
        base = s * nblk
        for p in range(passes):
            chunk = p * _NC + c

            def run_pass(chunk=chunk):
                pltpu.sync_copy(z_hbm, acc.at[pl.ds(r0, rpt)])
                plsc.subcore_barrier()
                xc = x_hbm.at[chunk]
                pltpu.async_copy(ei_hbm.at[base], ib0, semi0)

                def body(i, _):
                    g0 = base + 2 * i
                    pltpu.make_async_copy(ei_hbm.at[base], ib0, semi0).wait()
                    pltpu.async_copy(xc.at[ib0.at[0]], rows0, semg0)

                    @pl.when(i > 0)
                    def _prev():
                        pltpu.make_async_copy(xc.at[ib1.at[0]], rows1,
                                              semg1).wait()
                        pltpu.sync_copy(rows1, acc.at[ib1.at[1]], add=True)

                    pltpu.async_copy(ei_hbm.at[g0 + 1], ib1, semi1)
                    pltpu.make_async_copy(ei_hbm.at[base], ib1, semi1).wait()
                    pltpu.make_async_copy(xc.at[ib0.at[0]], rows0,
                                          semg0).wait()
                    pltpu.async_copy(xc.at[ib1.at[0]], rows1, semg1)
                    pltpu.sync_copy(rows0, acc.at[ib0.at[1]], add=True)

                    @pl.when(i + 1 < nhalf)
                    def _next():
                        pltpu.async_copy(ei_hbm.at[g0 + 2], ib0, semi0)

                    return _

                lax.fori_loop(0, nhalf, body, None)
                pltpu.make_async_copy(xc.at[ib1.at[0]], rows1, semg1).wait()
                pltpu.sync_copy(rows1, acc.at[ib1.at[1]], add=True)
                plsc.subcore_barrier()
                pltpu.sync_copy(acc.at[pl.ds(r0, rpt)],
                                out_hbm.at[chunk, pl.ds(r0, rpt)])
                plsc.subcore_barrier()

            if p * _NC + _NC <= nchunk:
                run_pass()
            else:
                pl.when(chunk < nchunk)(run_pass)

    return k(x, ei, zeros_slice)


def _sc_gather(tab, idx):
    n, dcols = tab.shape
    ni = idx.shape[0]
    per_w = ni // (_NC * _NS)
    nb = per_w // 128
    mesh = plsc.VectorSubcoreMesh(core_axis_name="c", subcore_axis_name="s")

    @functools.partial(
        pl.kernel,
        out_type=jax.ShapeDtypeStruct((ni, dcols), F32),
        mesh=mesh,
        scratch_types=[
            pltpu.VMEM((128,), jnp.int32),
            pltpu.VMEM((128, dcols), F32),
            pltpu.SemaphoreType.DMA,
        ],
    )
    def k(tab_hbm, idx_hbm, out_hbm, ibuf, rows, sem):
        c = lax.axis_index("c")
        s = lax.axis_index("s")
        wid = s * _NC + c
        for b in range(nb):
            o = pl.multiple_of(wid * per_w + b * 128, 128)
            pltpu.sync_copy(idx_hbm.at[pl.ds(o, 128)], ibuf)
            pltpu.async_copy(tab_hbm.at[ibuf], rows, sem).wait()
            pltpu.sync_copy(rows, out_hbm.at[pl.ds(o, 128)])

    return k(tab, idx)


def kernel(x_drugs, ddi_edge_attr, params, edge_index, ddi_edge_idx):
    nd, d = x_drugs.shape
    prot = params['prot_emb']
    npr = prot.shape[0]
    n = nd + npr
    c1, r0, r1 = params['conv1'], params['res0'], params['res1']
    zeros_slice = jnp.zeros(((-(-n // _NS) + 7) // 8 * 8, 128), F32)

    e = edge_index.shape[1]
    epad = -(-e // (_NS * 256)) * (_NS * 256)
    if epad != e:
        pn = epad - e
        pad_edges = jnp.stack([
            jnp.zeros((pn,), jnp.int32),
            n + (jnp.arange(pn, dtype=jnp.int32) % 8)])
        edge_index = jnp.concatenate([edge_index, pad_edges], axis=1)
    edge_index = edge_index.reshape(2, epad // 128, 128).transpose(1, 0, 2)

    def pad(a):
        return jnp.concatenate(
            [a, jnp.zeros((n - a.shape[0], a.shape[1]), F32)], axis=0)

    hd0 = pad(x_drugs)
    hp0 = jnp.concatenate([jnp.zeros((nd, d), F32), prot], axis=0)

    att0, xdp1 = _lra_call(x_drugs, params['lra0'], pre_w=c1['dp'])
    xdp1p = pad(xdp1)
    pw = _mm_call(prot, c1['pd'])
    pwp = jnp.concatenate([jnp.zeros((nd, d), F32), pw], axis=0)
    x1 = jnp.stack([xdp1p[:, :128], xdp1p[:, 128:],
                    pwp[:, :128], pwp[:, 128:]])
    s1 = _sc_scatter(x1, edge_index, zeros_slice, 4)
    hd1, hp1, x2 = _layer_call('first', hd0, hp0, s1, pad(att0),
                               wdd=c1['dd'], wpd=None, wpp=c1['pp'],
                               wdp=r0['dp'])

    s2 = _sc_scatter(x2, edge_index, zeros_slice, 4)
    att1 = _lra_call(hd1[:nd], params['lra1'])
    hd2, x3 = _layer_call('mid', hd1, hp1, s2, pad(att1),
                          wdd=r0['dd'], wpd=r0['pd'], wpp=r0['pp'])

    s3 = _sc_scatter(x3, edge_index, zeros_slice, 2)
    att2 = _lra_call(hd2[:nd], params['lra2'])
    hd3 = _layer_call('last', hd2, None, s3, pad(att2),
                      wdd=r1['dd'], wpd=r1['pd'])

    idx_flat = jnp.concatenate([ddi_edge_idx[:, 0], ddi_edge_idx[:, 1]])
    g = _sc_gather(hd3, idx_flat)
    w1 = params['pred']['W1']
    d3 = hd2.shape[1] + 2 * params['lra2']['u'].shape[1]
    dpad = hd3.shape[1] - d3
    wpad = lambda w: jnp.concatenate(
        [w, jnp.zeros((dpad, w.shape[1]), F32)], axis=0)
    return _pred_call(g, ddi_edge_attr,
                      wpad(w1[:d3]), wpad(w1[d3:2 * d3]), w1[2 * d3:],
                      params['pred']['b1'].reshape(1, -1),
                      params['pred']['W2'],
                      params['pred']['b2'].reshape(1, 1))
